# Initial kernel scaffold; baseline (speedup 1.0000x reference)
import jax
import jax.numpy as jnp
from jax.experimental import pallas as pl


def kernel(poi_embed_table, gcn_W, gcn_b, in_proj_w, in_proj_b, out_proj_w, out_proj_b, dist_vec, dist_edges, batch_idx, poi, x_idx):
    raise NotImplementedError("write your pallas kernel here")


# speedup vs baseline: 6.7008x; 6.7008x over previous
import functools

import jax
import jax.numpy as jnp
import numpy as np
from jax import lax
from jax.experimental import pallas as pl
from jax.experimental.pallas import tpu as pltpu
from jax.experimental.pallas import tpu_sc as plsc

N_POI = 10000
EMBED = 128
HEADS = 4
E_RAW = 320000
N_SESS = 256
SESS_LEN = 32

NC, NS = 2, 16
NW = NC * NS
NPAD = 10240
ROWS_PER_TILE = NPAD // NS
CHUNK = 128
E_TOT = 2 * E_RAW + N_POI
CPT = -(-E_TOT // (CHUNK * NW))
EPAD = CPT * NW * CHUNK
GIDX = N_SESS + N_SESS * SESS_LEN
GPT = GIDX // NW

def _mesh():
    return plsc.VectorSubcoreMesh(core_axis_name="c", subcore_axis_name="s",
                                  num_cores=NC, num_subcores=NS)


def _stripe(sid):
    return pl.ds(sid * ROWS_PER_TILE, ROWS_PER_TILE)


def _sc_deg_body(col_hbm, zeros128_hbm, ones128_hbm, deg_out, acc_sh, idx_v,
                 ones_v):
    cid = lax.axis_index("c")
    sid = lax.axis_index("s")
    wid = sid * NC + cid

    pltpu.sync_copy(zeros128_hbm.at[_stripe(sid)], acc_sh.at[_stripe(sid)])
    pltpu.sync_copy(ones128_hbm, ones_v)
    plsc.subcore_barrier()

    def chunk_body(k, carry):
        base = (wid * CPT + k) * CHUNK
        pltpu.sync_copy(col_hbm.at[pl.ds(base, CHUNK)], idx_v.at[0])
        pltpu.sync_copy(ones_v, acc_sh.at[idx_v.at[0]], add=True)
        return carry

    lax.fori_loop(0, CPT, chunk_body, 0)
    plsc.subcore_barrier()
    pltpu.sync_copy(acc_sh.at[_stripe(sid)], deg_out.at[cid, _stripe(sid)])


@functools.lru_cache(maxsize=None)
def _sc_deg_call():
    return pl.kernel(
        _sc_deg_body,
        out_type=jax.ShapeDtypeStruct((NC, NPAD, EMBED), jnp.float32),
        mesh=_mesh(),
        scratch_types=[
            pltpu.VMEM_SHARED((NPAD, EMBED), jnp.float32),
            pltpu.VMEM((1, CHUNK), jnp.int32),
            pltpu.VMEM((CHUNK, EMBED), jnp.float32),
        ],
    )


def _sc_deg(col, zeros128, ones128):
    return _sc_deg_call()(col, zeros128, ones128)


def _sc_agg_body(row_hbm, col_hbm, w16_hbm, tbl_hbm, zeros128_hbm, acc_out,
                 acc_sh, idxc_v, idxr_v, wrow_v, rows_v, sem):
    cid = lax.axis_index("c")
    sid = lax.axis_index("s")
    wid = sid * NC + cid

    pltpu.sync_copy(zeros128_hbm.at[_stripe(sid)], acc_sh.at[_stripe(sid)])
    plsc.subcore_barrier()

    def chunk_body(k, carry):
        base = (wid * CPT + k) * CHUNK
        pltpu.sync_copy(col_hbm.at[pl.ds(base, CHUNK)], idxc_v.at[0])
        pltpu.sync_copy(row_hbm.at[pl.ds(base, CHUNK)], idxr_v.at[0])
        pltpu.sync_copy(w16_hbm.at[pl.ds(base, CHUNK)], wrow_v)
        pltpu.async_copy(tbl_hbm.at[idxc_v.at[0]], rows_v, sem).wait()

        def e_body(e, ec):
            s = wrow_v[e]
            for j in range(EMBED // 16):
                sl = pl.ds(j * 16, 16)
                rows_v[e, sl] = rows_v[e, sl] * s
            return ec

        lax.fori_loop(0, CHUNK, e_body, 0)
        pltpu.sync_copy(rows_v, acc_sh.at[idxr_v.at[0]], add=True)
        return carry

    lax.fori_loop(0, CPT, chunk_body, 0)
    plsc.subcore_barrier()
    pltpu.sync_copy(acc_sh.at[_stripe(sid)], acc_out.at[cid, _stripe(sid)])


@functools.lru_cache(maxsize=None)
def _sc_agg_call():
    return pl.kernel(
        _sc_agg_body,
        out_type=jax.ShapeDtypeStruct((NC, NPAD, EMBED), jnp.float32),
        mesh=_mesh(),
        scratch_types=[
            pltpu.VMEM_SHARED((NPAD, EMBED), jnp.float32),
            pltpu.VMEM((1, CHUNK), jnp.int32),
            pltpu.VMEM((1, CHUNK), jnp.int32),
            pltpu.VMEM((CHUNK, 16), jnp.float32),
            pltpu.VMEM((CHUNK, EMBED), jnp.float32),
            pltpu.SemaphoreType.DMA,
        ],
    )


def _sc_agg(row, col, w16, tbl, zeros128):
    return _sc_agg_call()(row, col, w16, tbl, zeros128)


def _sc_gather_body(idx_hbm, tbl_hbm, rows_out, idx_v, rows_v, sem):
    cid = lax.axis_index("c")
    sid = lax.axis_index("s")
    wid = sid * NC + cid
    for off, cnt in ((0, CHUNK), (CHUNK, CHUNK), (2 * CHUNK, GPT - 2 * CHUNK)):
        base = wid * GPT + off
        pltpu.sync_copy(idx_hbm.at[pl.ds(base, cnt)], idx_v.at[0, pl.ds(0, cnt)])
        pltpu.async_copy(tbl_hbm.at[idx_v.at[0, pl.ds(0, cnt)]],
                         rows_v.at[pl.ds(0, cnt)], sem).wait()
        pltpu.sync_copy(rows_v.at[pl.ds(0, cnt)], rows_out.at[pl.ds(base, cnt)])


@functools.lru_cache(maxsize=None)
def _sc_gather_call():
    return pl.kernel(
        _sc_gather_body,
        out_type=jax.ShapeDtypeStruct((GIDX, EMBED), jnp.float32),
        mesh=_mesh(),
        scratch_types=[
            pltpu.VMEM((1, CHUNK), jnp.int32),
            pltpu.VMEM((CHUNK, EMBED), jnp.float32),
            pltpu.SemaphoreType.DMA,
        ],
    )


def _sc_gather(idx, tbl):
    return _sc_gather_call()(idx, tbl)


_WBLK = 4096


def _tc_wexp_body(dv, w_o):
    d = dv[...]
    w_o[...] = jnp.broadcast_to(jnp.exp(-d * d), (_WBLK, 16))


def _tc_wexp(dvp_col):
    return pl.pallas_call(
        _tc_wexp_body,
        grid=(EPAD // _WBLK,),
        in_specs=[pl.BlockSpec((_WBLK, 1), lambda i: (i, 0))],
        out_specs=pl.BlockSpec((_WBLK, 16), lambda i: (i, 0)),
        out_shape=jax.ShapeDtypeStruct((EPAD, 16), jnp.float32),
    )(dvp_col)


_ROWS_B = 512
_NBLK = NPAD // _ROWS_B


def _tc_prep_body(degp, emb, dis_o, encp_o):
    deg = degp[0, :, 0:1] + degp[1, :, 0:1]
    dis = jnp.where(deg > 0, lax.rsqrt(deg), 0.0)
    disb = jnp.broadcast_to(dis, (_ROWS_B, EMBED))
    dis_o[...] = disb
    encp_o[...] = disb * emb[...]


def _tc_prep(deg_parts, emb_pad):
    return pl.pallas_call(
        _tc_prep_body,
        grid=(_NBLK,),
        in_specs=[
            pl.BlockSpec((NC, _ROWS_B, EMBED), lambda i: (0, i, 0)),
            pl.BlockSpec((_ROWS_B, EMBED), lambda i: (i, 0)),
        ],
        out_specs=[
            pl.BlockSpec((_ROWS_B, EMBED), lambda i: (i, 0)),
            pl.BlockSpec((_ROWS_B, EMBED), lambda i: (i, 0)),
        ],
        out_shape=[
            jax.ShapeDtypeStruct((NPAD, EMBED), jnp.float32),
            jax.ShapeDtypeStruct((NPAD, EMBED), jnp.float32),
        ],
    )(deg_parts, emb_pad)


def _tc_dense_body(accp, disb, W, b, enc_o, encs_o):
    side = (accp[0] + accp[1]) * disb[...]
    out = lax.dot_general(side, W[...], (((1,), (1,)), ((), ())),
                          preferred_element_type=jnp.float32) + b[...]
    out = jnp.where(out >= 0, out, 0.01 * out)
    nrm = jnp.sqrt(jnp.sum(out * out, axis=1, keepdims=True))
    enc = out / jnp.maximum(nrm, 1e-12)
    enc_o[...] = enc
    encs_o[...] = enc * disb[...]


def _tc_dense(acc_parts, dis_b, W, b):
    return pl.pallas_call(
        _tc_dense_body,
        grid=(_NBLK,),
        in_specs=[
            pl.BlockSpec((NC, _ROWS_B, EMBED), lambda i: (0, i, 0)),
            pl.BlockSpec((_ROWS_B, EMBED), lambda i: (i, 0)),
            pl.BlockSpec((EMBED, EMBED), lambda i: (0, 0)),
            pl.BlockSpec((1, EMBED), lambda i: (0, 0)),
        ],
        out_specs=[
            pl.BlockSpec((_ROWS_B, EMBED), lambda i: (i, 0)),
            pl.BlockSpec((_ROWS_B, EMBED), lambda i: (i, 0)),
        ],
        out_shape=[
            jax.ShapeDtypeStruct((NPAD, EMBED), jnp.float32),
            jax.ShapeDtypeStruct((NPAD, EMBED), jnp.float32),
        ],
    )(acc_parts, dis_b, W, b)


_SB = 8
_TB = _SB * SESS_LEN
_HD = EMBED // HEADS


def _tc_attn_body(x, wqkv, bqkv, wo, bo, out_o):
    xb = x[...]
    qkv = lax.dot_general(xb, wqkv[...], (((1,), (1,)), ((), ())),
                          preferred_element_type=jnp.float32) + bqkv[...]
    r2 = lax.broadcasted_iota(jnp.int32, (_TB, _TB), 0) // SESS_LEN
    c2 = lax.broadcasted_iota(jnp.int32, (_TB, _TB), 1) // SESS_LEN
    same = r2 == c2
    pieces = []
    for h in range(HEADS):
        q = qkv[:, h * _HD:(h + 1) * _HD]
        k = qkv[:, EMBED + h * _HD:EMBED + (h + 1) * _HD]
        v = qkv[:, 2 * EMBED + h * _HD:2 * EMBED + (h + 1) * _HD]
        s = lax.dot_general(q, k, (((1,), (1,)), ((), ())),
                            preferred_element_type=jnp.float32)
        s = s * np.float32(1.0 / np.sqrt(_HD))
        s = jnp.where(same, s, -1e30)
        m = jnp.max(s, axis=1, keepdims=True)
        p = jnp.exp(s - m)
        p = p / jnp.sum(p, axis=1, keepdims=True)
        pieces.append(lax.dot_general(p, v, (((1,), (0,)), ((), ())),
                                      preferred_element_type=jnp.float32))
    ctx = jnp.concatenate(pieces, axis=1)
    ao = lax.dot_general(ctx, wo[...], (((1,), (1,)), ((), ())),
                         preferred_element_type=jnp.float32) + bo[...]
    rm = lax.broadcasted_iota(jnp.int32, (_SB, _TB), 0)
    cm = lax.broadcasted_iota(jnp.int32, (_SB, _TB), 1) // SESS_LEN
    Mm = jnp.where(cm == rm, np.float32(1.0 / SESS_LEN), 0.0)
    out_o[...] = lax.dot_general(Mm, ao, (((1,), (0,)), ((), ())),
                                 preferred_element_type=jnp.float32)


def _tc_attn(geo, wqkv, bqkv, wo, bo):
    return pl.pallas_call(
        _tc_attn_body,
        grid=(N_SESS // _SB,),
        in_specs=[
            pl.BlockSpec((_TB, EMBED), lambda i: (i, 0)),
            pl.BlockSpec((3 * EMBED, EMBED), lambda i: (0, 0)),
            pl.BlockSpec((1, 3 * EMBED), lambda i: (0, 0)),
            pl.BlockSpec((EMBED, EMBED), lambda i: (0, 0)),
            pl.BlockSpec((1, EMBED), lambda i: (0, 0)),
        ],
        out_specs=pl.BlockSpec((_SB, EMBED), lambda i: (i, 0)),
        out_shape=jax.ShapeDtypeStruct((N_SESS, EMBED), jnp.float32),
    )(geo, wqkv, bqkv, wo, bo)


def kernel(poi_embed_table, gcn_W, gcn_b, in_proj_w, in_proj_b, out_proj_w,
           out_proj_b, dist_vec, dist_edges, batch_idx, poi, x_idx):
    i32 = jnp.int32
    f32 = jnp.float32
    loop = jnp.arange(N_POI, dtype=dist_edges.dtype)
    edges = jnp.concatenate(
        [dist_edges, dist_edges[::-1], jnp.stack([loop, loop])], axis=1)
    dvec_all = jnp.concatenate(
        [dist_vec, dist_vec, jnp.zeros((N_POI,), f32)])
    pad = EPAD - E_TOT
    trash = jnp.full((pad,), NPAD - 1, i32)
    row = jnp.concatenate([edges[0].astype(i32), trash])
    col = jnp.concatenate([edges[1].astype(i32), trash])
    dvp = jnp.concatenate([dvec_all, jnp.zeros((pad,), f32)])
    emb_pad = jnp.pad(poi_embed_table, ((0, NPAD - N_POI), (0, 0)))
    zeros128 = jnp.zeros((NPAD, EMBED), f32)
    w16 = _tc_wexp(dvp.reshape(EPAD, 1))
    ones128 = jnp.ones((CHUNK, EMBED), f32)
    deg_parts = _sc_deg(col, zeros128, ones128)
    dis_b, enc0p = _tc_prep(deg_parts, emb_pad)
    acc1 = _sc_agg(row, col, w16, enc0p, zeros128)
    enc1, enc1s = _tc_dense(acc1, dis_b, gcn_W[0], gcn_b[0].reshape(1, -1))
    acc2 = _sc_agg(row, col, w16, enc1s, zeros128)
    enc2, _ = _tc_dense(acc2, dis_b, gcn_W[1], gcn_b[1].reshape(1, -1))

    gidx = jnp.concatenate([poi.astype(i32), x_idx.astype(i32)])
    rows = _sc_gather(gidx, enc2)
    tar_embed = rows[:N_SESS]
    geo = rows[N_SESS:]
    aggr_feat = _tc_attn(geo, in_proj_w, in_proj_b.reshape(1, -1),
                         out_proj_w, out_proj_b.reshape(1, -1))
    return aggr_feat, tar_embed


# speedup vs baseline: 7.8329x; 1.1689x over previous
import functools

import jax
import jax.numpy as jnp
import numpy as np
from jax import lax
from jax.experimental import pallas as pl
from jax.experimental.pallas import tpu as pltpu
from jax.experimental.pallas import tpu_sc as plsc

N_POI = 10000
EMBED = 128
HEADS = 4
E_RAW = 320000
N_SESS = 256
SESS_LEN = 32

NC, NS = 2, 16
NW = NC * NS
NPAD = 10112
ROWS_PER_TILE = NPAD // NS
CHUNK = 64
CPT = -(-E_RAW // (CHUNK * NW))
EPAD = CPT * NW * CHUNK
GIDX = N_SESS + N_SESS * SESS_LEN
GPT = GIDX // NW


def _mesh():
    return plsc.VectorSubcoreMesh(core_axis_name="c", subcore_axis_name="s",
                                  num_cores=NC, num_subcores=NS)


def _stripe(sid):
    return pl.ds(sid * ROWS_PER_TILE, ROWS_PER_TILE)


def _sc_deg_body(e0_hbm, e1_hbm, zeros128_hbm, ones128_hbm, deg_out, acc_sh,
                 idx_v, ones_v):
    cid = lax.axis_index("c")
    sid = lax.axis_index("s")
    wid = sid * NC + cid

    pltpu.sync_copy(zeros128_hbm.at[_stripe(sid)], acc_sh.at[_stripe(sid)])
    pltpu.sync_copy(ones128_hbm, ones_v)
    plsc.subcore_barrier()

    def chunk_body(k, carry):
        base = (wid * CPT + k) * CHUNK
        pltpu.sync_copy(e0_hbm.at[pl.ds(base, CHUNK)], idx_v.at[0])
        pltpu.sync_copy(e1_hbm.at[pl.ds(base, CHUNK)], idx_v.at[1])
        pltpu.sync_copy(ones_v, acc_sh.at[idx_v.at[0]], add=True)
        pltpu.sync_copy(ones_v, acc_sh.at[idx_v.at[1]], add=True)
        return carry

    lax.fori_loop(0, CPT, chunk_body, 0)
    plsc.subcore_barrier()
    pltpu.sync_copy(acc_sh.at[_stripe(sid)], deg_out.at[cid, _stripe(sid)])


@functools.lru_cache(maxsize=None)
def _sc_deg_call():
    return pl.kernel(
        _sc_deg_body,
        out_type=jax.ShapeDtypeStruct((NC, NPAD, EMBED), jnp.float32),
        mesh=_mesh(),
        scratch_types=[
            pltpu.VMEM_SHARED((NPAD, EMBED), jnp.float32),
            pltpu.VMEM((2, CHUNK), jnp.int32),
            pltpu.VMEM((CHUNK, EMBED), jnp.float32),
        ],
    )


def _sc_deg(e0, e1, zeros128, ones128):
    return _sc_deg_call()(e0, e1, zeros128, ones128)


def _sc_agg_body(e0_hbm, e1_hbm, w16_hbm, tbl_hbm, zeros128_hbm, acc_out,
                 acc_sh, idx_v, w_v, rows_v, gsem, ssem, isem):
    cid = lax.axis_index("c")
    sid = lax.axis_index("s")
    wid = sid * NC + cid

    pltpu.sync_copy(zeros128_hbm.at[_stripe(sid)], acc_sh.at[_stripe(sid)])
    plsc.subcore_barrier()

    def cp_idx_async(k, b):
        base = (wid * CPT + k) * CHUNK
        pltpu.async_copy(e0_hbm.at[pl.ds(base, CHUNK)], idx_v.at[b, 0], isem)
        pltpu.async_copy(e1_hbm.at[pl.ds(base, CHUNK)], idx_v.at[b, 1], isem)
        pltpu.async_copy(w16_hbm.at[pl.ds(base, CHUNK)], w_v.at[b], isem)

    def drain_idx():
        pltpu.make_async_copy(e0_hbm.at[pl.ds(0, CHUNK)], idx_v.at[0, 0],
                              isem).wait()
        pltpu.make_async_copy(e1_hbm.at[pl.ds(0, CHUNK)], idx_v.at[0, 1],
                              isem).wait()
        pltpu.make_async_copy(w16_hbm.at[pl.ds(0, CHUNK)], w_v.at[0],
                              isem).wait()

    def issue_gather(s):
        k = lax.div(s, 2)
        d = lax.rem(s, 2)
        pltpu.async_copy(tbl_hbm.at[idx_v.at[lax.rem(k, 2), 1 - d]],
                         rows_v.at[lax.rem(s, 2)], gsem)

    def drain_scatter():
        pltpu.make_async_copy(rows_v.at[0], acc_sh.at[idx_v.at[0, 0]],
                              ssem).wait()

    cp_idx_async(0, 0)
    drain_idx()
    issue_gather(0)

    S = 2 * CPT

    def loop_body(s, carry):
        k = lax.div(s, 2)
        d = lax.rem(s, 2)
        b = lax.rem(s, 2)
        bk = lax.rem(k, 2)

        @pl.when(s > 0)
        def _():
            drain_scatter()

        @pl.when((d == 1) & (k + 1 < CPT))
        def _():
            drain_idx()

        @pl.when(s + 1 < S)
        def _():
            issue_gather(s + 1)

        pltpu.make_async_copy(tbl_hbm.at[idx_v.at[bk, 1 - d]],
                              rows_v.at[b], gsem).wait()

        def e_body(e, ec):
            sc = w_v[bk, e]
            for j in range(EMBED // 16):
                sl = pl.ds(j * 16, 16)
                rows_v[b, e, sl] = rows_v[b, e, sl] * sc
            return ec

        lax.fori_loop(0, CHUNK, e_body, 0)
        pltpu.async_copy(rows_v.at[b], acc_sh.at[idx_v.at[bk, d]], ssem,
                         add=True)

        @pl.when((d == 0) & (k + 1 < CPT))
        def _():
            cp_idx_async(k + 1, lax.rem(k + 1, 2))

        return carry

    lax.fori_loop(0, S, loop_body, 0)
    drain_scatter()
    plsc.subcore_barrier()
    pltpu.sync_copy(acc_sh.at[_stripe(sid)], acc_out.at[cid, _stripe(sid)])


@functools.lru_cache(maxsize=None)
def _sc_agg_call():
    return pl.kernel(
        _sc_agg_body,
        out_type=jax.ShapeDtypeStruct((NC, NPAD, EMBED), jnp.float32),
        mesh=_mesh(),
        scratch_types=[
            pltpu.VMEM_SHARED((NPAD, EMBED), jnp.float32),
            pltpu.VMEM((2, 2, CHUNK), jnp.int32),
            pltpu.VMEM((2, CHUNK, 16), jnp.float32),
            pltpu.VMEM((2, CHUNK, EMBED), jnp.float32),
            pltpu.SemaphoreType.DMA,
            pltpu.SemaphoreType.DMA,
            pltpu.SemaphoreType.DMA,
        ],
    )


def _sc_agg(e0, e1, w16, tbl, zeros128):
    return _sc_agg_call()(e0, e1, w16, tbl, zeros128)


def _sc_gather_body(idx_hbm, tbl_hbm, rows_out, idx_v, rows_v, sem):
    cid = lax.axis_index("c")
    sid = lax.axis_index("s")
    wid = sid * NC + cid
    for off, cnt in ((0, 128), (128, 128), (256, GPT - 256)):
        base = wid * GPT + off
        pltpu.sync_copy(idx_hbm.at[pl.ds(base, cnt)], idx_v.at[0, pl.ds(0, cnt)])
        pltpu.async_copy(tbl_hbm.at[idx_v.at[0, pl.ds(0, cnt)]],
                         rows_v.at[pl.ds(0, cnt)], sem).wait()
        pltpu.sync_copy(rows_v.at[pl.ds(0, cnt)], rows_out.at[pl.ds(base, cnt)])


@functools.lru_cache(maxsize=None)
def _sc_gather_call():
    return pl.kernel(
        _sc_gather_body,
        out_type=jax.ShapeDtypeStruct((GIDX, EMBED), jnp.float32),
        mesh=_mesh(),
        scratch_types=[
            pltpu.VMEM((1, 128), jnp.int32),
            pltpu.VMEM((128, EMBED), jnp.float32),
            pltpu.SemaphoreType.DMA,
        ],
    )


def _sc_gather(idx, tbl):
    return _sc_gather_call()(idx, tbl)


_WBLK = 2048


def _tc_wexp_body(dv, w_o):
    d = dv[...]
    w_o[...] = jnp.broadcast_to(jnp.exp(-d * d), (_WBLK, 16))


def _tc_wexp(dvp_col):
    return pl.pallas_call(
        _tc_wexp_body,
        grid=(EPAD // _WBLK,),
        in_specs=[pl.BlockSpec((_WBLK, 1), lambda i: (i, 0))],
        out_specs=pl.BlockSpec((_WBLK, 16), lambda i: (i, 0)),
        out_shape=jax.ShapeDtypeStruct((EPAD, 16), jnp.float32),
    )(dvp_col)


_ROWS_B = 632
_NBLK = NPAD // _ROWS_B


def _tc_prep_body(degp, emb, dis_o, encp_o):
    deg = degp[0, :, 0:1] + degp[1, :, 0:1] + 1.0
    dis = lax.rsqrt(deg)
    disb = jnp.broadcast_to(dis, (_ROWS_B, EMBED))
    dis_o[...] = disb
    encp_o[...] = disb * emb[...]


def _tc_prep(deg_parts, emb_pad):
    return pl.pallas_call(
        _tc_prep_body,
        grid=(_NBLK,),
        in_specs=[
            pl.BlockSpec((NC, _ROWS_B, EMBED), lambda i: (0, i, 0)),
            pl.BlockSpec((_ROWS_B, EMBED), lambda i: (i, 0)),
        ],
        out_specs=[
            pl.BlockSpec((_ROWS_B, EMBED), lambda i: (i, 0)),
            pl.BlockSpec((_ROWS_B, EMBED), lambda i: (i, 0)),
        ],
        out_shape=[
            jax.ShapeDtypeStruct((NPAD, EMBED), jnp.float32),
            jax.ShapeDtypeStruct((NPAD, EMBED), jnp.float32),
        ],
    )(deg_parts, emb_pad)


def _tc_dense_body(accp, tblprev, disb, W, b, enc_o, encs_o):
    side = (accp[0] + accp[1] + tblprev[...]) * disb[...]
    out = lax.dot_general(side, W[...], (((1,), (1,)), ((), ())),
                          preferred_element_type=jnp.float32) + b[...]
    out = jnp.where(out >= 0, out, 0.01 * out)
    nrm = jnp.sqrt(jnp.sum(out * out, axis=1, keepdims=True))
    enc = out / jnp.maximum(nrm, 1e-12)
    enc_o[...] = enc
    encs_o[...] = enc * disb[...]


def _tc_dense(acc_parts, tblprev, dis_b, W, b):
    return pl.pallas_call(
        _tc_dense_body,
        grid=(_NBLK,),
        in_specs=[
            pl.BlockSpec((NC, _ROWS_B, EMBED), lambda i: (0, i, 0)),
            pl.BlockSpec((_ROWS_B, EMBED), lambda i: (i, 0)),
            pl.BlockSpec((_ROWS_B, EMBED), lambda i: (i, 0)),
            pl.BlockSpec((EMBED, EMBED), lambda i: (0, 0)),
            pl.BlockSpec((1, EMBED), lambda i: (0, 0)),
        ],
        out_specs=[
            pl.BlockSpec((_ROWS_B, EMBED), lambda i: (i, 0)),
            pl.BlockSpec((_ROWS_B, EMBED), lambda i: (i, 0)),
        ],
        out_shape=[
            jax.ShapeDtypeStruct((NPAD, EMBED), jnp.float32),
            jax.ShapeDtypeStruct((NPAD, EMBED), jnp.float32),
        ],
    )(acc_parts, tblprev, dis_b, W, b)


_SB = 8
_TB = _SB * SESS_LEN
_HD = EMBED // HEADS


def _tc_attn_body(x, wqkv, bqkv, wo, bo, out_o):
    xb = x[...]
    qkv = lax.dot_general(xb, wqkv[...], (((1,), (1,)), ((), ())),
                          preferred_element_type=jnp.float32) + bqkv[...]
    r2 = lax.broadcasted_iota(jnp.int32, (_TB, _TB), 0) // SESS_LEN
    c2 = lax.broadcasted_iota(jnp.int32, (_TB, _TB), 1) // SESS_LEN
    same = r2 == c2
    pieces = []
    for h in range(HEADS):
        q = qkv[:, h * _HD:(h + 1) * _HD]
        k = qkv[:, EMBED + h * _HD:EMBED + (h + 1) * _HD]
        v = qkv[:, 2 * EMBED + h * _HD:2 * EMBED + (h + 1) * _HD]
        s = lax.dot_general(q, k, (((1,), (1,)), ((), ())),
                            preferred_element_type=jnp.float32)
        s = s * np.float32(1.0 / np.sqrt(_HD))
        s = jnp.where(same, s, -1e30)
        m = jnp.max(s, axis=1, keepdims=True)
        p = jnp.exp(s - m)
        p = p / jnp.sum(p, axis=1, keepdims=True)
        pieces.append(lax.dot_general(p, v, (((1,), (0,)), ((), ())),
                                      preferred_element_type=jnp.float32))
    ctx = jnp.concatenate(pieces, axis=1)
    ao = lax.dot_general(ctx, wo[...], (((1,), (1,)), ((), ())),
                         preferred_element_type=jnp.float32) + bo[...]
    rm = lax.broadcasted_iota(jnp.int32, (_SB, _TB), 0)
    cm = lax.broadcasted_iota(jnp.int32, (_SB, _TB), 1) // SESS_LEN
    Mm = jnp.where(cm == rm, np.float32(1.0 / SESS_LEN), 0.0)
    out_o[...] = lax.dot_general(Mm, ao, (((1,), (0,)), ((), ())),
                                 preferred_element_type=jnp.float32)


def _tc_attn(geo, wqkv, bqkv, wo, bo):
    return pl.pallas_call(
        _tc_attn_body,
        grid=(N_SESS // _SB,),
        in_specs=[
            pl.BlockSpec((_TB, EMBED), lambda i: (i, 0)),
            pl.BlockSpec((3 * EMBED, EMBED), lambda i: (0, 0)),
            pl.BlockSpec((1, 3 * EMBED), lambda i: (0, 0)),
            pl.BlockSpec((EMBED, EMBED), lambda i: (0, 0)),
            pl.BlockSpec((1, EMBED), lambda i: (0, 0)),
        ],
        out_specs=pl.BlockSpec((_SB, EMBED), lambda i: (i, 0)),
        out_shape=jax.ShapeDtypeStruct((N_SESS, EMBED), jnp.float32),
    )(geo, wqkv, bqkv, wo, bo)


def kernel(poi_embed_table, gcn_W, gcn_b, in_proj_w, in_proj_b, out_proj_w,
           out_proj_b, dist_vec, dist_edges, batch_idx, poi, x_idx):
    i32 = jnp.int32
    f32 = jnp.float32
    pad = EPAD - E_RAW
    trash = jnp.full((pad,), NPAD - 1, i32)
    e0 = jnp.concatenate([dist_edges[0].astype(i32), trash])
    e1 = jnp.concatenate([dist_edges[1].astype(i32), trash])
    dvp = jnp.concatenate([dist_vec, jnp.zeros((pad,), f32)])
    emb_pad = jnp.pad(poi_embed_table, ((0, NPAD - N_POI), (0, 0)))
    zeros128 = jnp.zeros((NPAD, EMBED), f32)
    ones128 = jnp.ones((CHUNK, EMBED), f32)

    w16 = _tc_wexp(dvp.reshape(EPAD, 1))
    deg_parts = _sc_deg(e0, e1, zeros128, ones128)
    dis_b, enc0p = _tc_prep(deg_parts, emb_pad)
    acc1 = _sc_agg(e0, e1, w16, enc0p, zeros128)
    enc1, enc1s = _tc_dense(acc1, enc0p, dis_b, gcn_W[0],
                            gcn_b[0].reshape(1, -1))
    acc2 = _sc_agg(e0, e1, w16, enc1s, zeros128)
    enc2, _ = _tc_dense(acc2, enc1s, dis_b, gcn_W[1], gcn_b[1].reshape(1, -1))

    gidx = jnp.concatenate([poi.astype(i32), x_idx.astype(i32)])
    rows = _sc_gather(gidx, enc2)
    tar_embed = rows[:N_SESS]
    geo = rows[N_SESS:]
    aggr_feat = _tc_attn(geo, in_proj_w, in_proj_b.reshape(1, -1),
                         out_proj_w, out_proj_b.reshape(1, -1))
    return aggr_feat, tar_embed


# speedup vs baseline: 12.5453x; 1.6016x over previous
import functools

import jax
import jax.numpy as jnp
import numpy as np
from jax import lax
from jax.experimental import pallas as pl
from jax.experimental.pallas import tpu as pltpu
from jax.experimental.pallas import tpu_sc as plsc

N_POI = 10000
EMBED = 128
HEADS = 4
E_RAW = 320000
N_SESS = 256
SESS_LEN = 32

NC, NS = 2, 16
NW = NC * NS
NPAD = 10112
ROWS_PER_TILE = NPAD // NS
CHUNK = 128
CPT = -(-E_RAW // (CHUNK * NW))
EPAD = CPT * NW * CHUNK
GIDX = N_SESS + N_SESS * SESS_LEN
GPT = GIDX // NW


def _mesh():
    return plsc.VectorSubcoreMesh(core_axis_name="c", subcore_axis_name="s",
                                  num_cores=NC, num_subcores=NS)


def _stripe(sid):
    return pl.ds(sid * ROWS_PER_TILE, ROWS_PER_TILE)


def _sc_deg_body(e0_hbm, e1_hbm, zeros128_hbm, ones128_hbm, deg_out, acc_sh,
                 idx_v, ones_v):
    cid = lax.axis_index("c")
    sid = lax.axis_index("s")
    wid = sid * NC + cid

    pltpu.sync_copy(zeros128_hbm.at[_stripe(sid)], acc_sh.at[_stripe(sid)])
    pltpu.sync_copy(ones128_hbm, ones_v)
    plsc.subcore_barrier()

    def chunk_body(k, carry):
        base = (wid * CPT + k) * CHUNK
        pltpu.sync_copy(e0_hbm.at[pl.ds(base, CHUNK)], idx_v.at[0])
        pltpu.sync_copy(e1_hbm.at[pl.ds(base, CHUNK)], idx_v.at[1])
        pltpu.sync_copy(ones_v, acc_sh.at[idx_v.at[0]], add=True)
        pltpu.sync_copy(ones_v, acc_sh.at[idx_v.at[1]], add=True)
        return carry

    lax.fori_loop(0, CPT, chunk_body, 0)
    plsc.subcore_barrier()
    pltpu.sync_copy(acc_sh.at[_stripe(sid)], deg_out.at[cid, _stripe(sid)])


@functools.lru_cache(maxsize=None)
def _sc_deg_call():
    return pl.kernel(
        _sc_deg_body,
        out_type=jax.ShapeDtypeStruct((NC, NPAD, EMBED), jnp.float32),
        mesh=_mesh(),
        scratch_types=[
            pltpu.VMEM_SHARED((NPAD, EMBED), jnp.float32),
            pltpu.VMEM((2, CHUNK), jnp.int32),
            pltpu.VMEM((CHUNK, EMBED), jnp.float32),
        ],
    )


def _sc_deg(e0, e1, zeros128, ones128):
    return _sc_deg_call()(e0, e1, zeros128, ones128)


def _sc_agg_body(e0_hbm, e1_hbm, wflat_hbm, tbl_hbm, zeros128_hbm, acc_out,
                 acc_sh, idx_v, w_v, rows_v, gsem, ssem, isem):
    cid = lax.axis_index("c")
    sid = lax.axis_index("s")
    wid = sid * NC + cid

    pltpu.sync_copy(zeros128_hbm.at[_stripe(sid)], acc_sh.at[_stripe(sid)])
    plsc.subcore_barrier()

    def cp_idx_async(k, b):
        base = (wid * CPT + k) * CHUNK
        pltpu.async_copy(e0_hbm.at[pl.ds(base, CHUNK)], idx_v.at[b, 0], isem)
        pltpu.async_copy(e1_hbm.at[pl.ds(base, CHUNK)], idx_v.at[b, 1], isem)
        pltpu.async_copy(wflat_hbm.at[pl.ds(base * 16, CHUNK * 16)],
                         w_v.at[b], isem)

    def drain_idx():
        pltpu.make_async_copy(e0_hbm.at[pl.ds(0, CHUNK)], idx_v.at[0, 0],
                              isem).wait()
        pltpu.make_async_copy(e1_hbm.at[pl.ds(0, CHUNK)], idx_v.at[0, 1],
                              isem).wait()
        pltpu.make_async_copy(wflat_hbm.at[pl.ds(0, CHUNK * 16)], w_v.at[0],
                              isem).wait()

    def issue_gather(s):
        k = lax.div(s, 2)
        d = lax.rem(s, 2)
        pltpu.async_copy(tbl_hbm.at[idx_v.at[lax.rem(k, 2), 1 - d]],
                         rows_v.at[lax.rem(s, 2)], gsem)

    def drain_scatter():
        pltpu.make_async_copy(rows_v.at[0], acc_sh.at[idx_v.at[0, 0]],
                              ssem).wait()

    cp_idx_async(0, 0)
    drain_idx()
    issue_gather(0)

    S = 2 * CPT

    def loop_body(s, carry):
        k = lax.div(s, 2)
        d = lax.rem(s, 2)
        b = lax.rem(s, 2)
        bk = lax.rem(k, 2)

        @pl.when(s > 0)
        def _():
            drain_scatter()

        @pl.when((d == 1) & (k + 1 < CPT))
        def _():
            drain_idx()

        @pl.when(s + 1 < S)
        def _():
            issue_gather(s + 1)

        pltpu.make_async_copy(tbl_hbm.at[idx_v.at[bk, 1 - d]],
                              rows_v.at[b], gsem).wait()

        def e_body(e, ec):
            sc = w_v[bk, pl.ds(e * 16, 16)]
            for j in range(EMBED // 16):
                sl = pl.ds(j * 16, 16)
                rows_v[b, e, sl] = rows_v[b, e, sl] * sc
            return ec

        lax.fori_loop(0, CHUNK, e_body, 0)
        pltpu.async_copy(rows_v.at[b], acc_sh.at[idx_v.at[bk, d]], ssem,
                         add=True)

        @pl.when((d == 0) & (k + 1 < CPT))
        def _():
            cp_idx_async(k + 1, lax.rem(k + 1, 2))

        return carry

    lax.fori_loop(0, S, loop_body, 0)
    drain_scatter()
    plsc.subcore_barrier()
    pltpu.sync_copy(acc_sh.at[_stripe(sid)], acc_out.at[cid, _stripe(sid)])


@functools.lru_cache(maxsize=None)
def _sc_agg_call():
    return pl.kernel(
        _sc_agg_body,
        out_type=jax.ShapeDtypeStruct((NC, NPAD, EMBED), jnp.float32),
        mesh=_mesh(),
        scratch_types=[
            pltpu.VMEM_SHARED((NPAD, EMBED), jnp.float32),
            pltpu.VMEM((2, 2, CHUNK), jnp.int32),
            pltpu.VMEM((2, CHUNK * 16), jnp.float32),
            pltpu.VMEM((2, CHUNK, EMBED), jnp.float32),
            pltpu.SemaphoreType.DMA,
            pltpu.SemaphoreType.DMA,
            pltpu.SemaphoreType.DMA,
        ],
    )


def _sc_agg(e0, e1, w16, tbl, zeros128):
    return _sc_agg_call()(e0, e1, w16.reshape(EPAD * 16), tbl, zeros128)


def _sc_gather_body(idx_hbm, tbl_hbm, rows_out, idx_v, rows_v, sem):
    cid = lax.axis_index("c")
    sid = lax.axis_index("s")
    wid = sid * NC + cid
    for off, cnt in ((0, 128), (128, 128), (256, GPT - 256)):
        base = wid * GPT + off
        pltpu.sync_copy(idx_hbm.at[pl.ds(base, cnt)], idx_v.at[0, pl.ds(0, cnt)])
        pltpu.async_copy(tbl_hbm.at[idx_v.at[0, pl.ds(0, cnt)]],
                         rows_v.at[pl.ds(0, cnt)], sem).wait()
        pltpu.sync_copy(rows_v.at[pl.ds(0, cnt)], rows_out.at[pl.ds(base, cnt)])


@functools.lru_cache(maxsize=None)
def _sc_gather_call():
    return pl.kernel(
        _sc_gather_body,
        out_type=jax.ShapeDtypeStruct((GIDX, EMBED), jnp.float32),
        mesh=_mesh(),
        scratch_types=[
            pltpu.VMEM((1, 128), jnp.int32),
            pltpu.VMEM((128, EMBED), jnp.float32),
            pltpu.SemaphoreType.DMA,
        ],
    )


def _sc_gather(idx, tbl):
    return _sc_gather_call()(idx, tbl)


_WBLK = 4096


def _tc_wexp_body(dv, w_o):
    d = dv[...]
    w_o[...] = jnp.broadcast_to(jnp.exp(-d * d), (_WBLK, 16))


def _tc_wexp(dvp_col):
    return pl.pallas_call(
        _tc_wexp_body,
        grid=(EPAD // _WBLK,),
        in_specs=[pl.BlockSpec((_WBLK, 1), lambda i: (i, 0))],
        out_specs=pl.BlockSpec((_WBLK, 16), lambda i: (i, 0)),
        out_shape=jax.ShapeDtypeStruct((EPAD, 16), jnp.float32),
    )(dvp_col)


_ROWS_B = 632
_NBLK = NPAD // _ROWS_B


def _tc_prep_body(degp, emb, dis_o, encp_o):
    deg = degp[0, :, 0:1] + degp[1, :, 0:1] + 1.0
    dis = lax.rsqrt(deg)
    disb = jnp.broadcast_to(dis, (_ROWS_B, EMBED))
    dis_o[...] = disb
    encp_o[...] = disb * emb[...]


def _tc_prep(deg_parts, emb_pad):
    return pl.pallas_call(
        _tc_prep_body,
        grid=(_NBLK,),
        in_specs=[
            pl.BlockSpec((NC, _ROWS_B, EMBED), lambda i: (0, i, 0)),
            pl.BlockSpec((_ROWS_B, EMBED), lambda i: (i, 0)),
        ],
        out_specs=[
            pl.BlockSpec((_ROWS_B, EMBED), lambda i: (i, 0)),
            pl.BlockSpec((_ROWS_B, EMBED), lambda i: (i, 0)),
        ],
        out_shape=[
            jax.ShapeDtypeStruct((NPAD, EMBED), jnp.float32),
            jax.ShapeDtypeStruct((NPAD, EMBED), jnp.float32),
        ],
    )(deg_parts, emb_pad)


def _tc_dense_body(accp, tblprev, disb, W, b, enc_o, encs_o):
    side = (accp[0] + accp[1] + tblprev[...]) * disb[...]
    out = lax.dot_general(side, W[...], (((1,), (1,)), ((), ())),
                          preferred_element_type=jnp.float32) + b[...]
    out = jnp.where(out >= 0, out, 0.01 * out)
    nrm = jnp.sqrt(jnp.sum(out * out, axis=1, keepdims=True))
    enc = out / jnp.maximum(nrm, 1e-12)
    enc_o[...] = enc
    encs_o[...] = enc * disb[...]


def _tc_dense(acc_parts, tblprev, dis_b, W, b):
    return pl.pallas_call(
        _tc_dense_body,
        grid=(_NBLK,),
        in_specs=[
            pl.BlockSpec((NC, _ROWS_B, EMBED), lambda i: (0, i, 0)),
            pl.BlockSpec((_ROWS_B, EMBED), lambda i: (i, 0)),
            pl.BlockSpec((_ROWS_B, EMBED), lambda i: (i, 0)),
            pl.BlockSpec((EMBED, EMBED), lambda i: (0, 0)),
            pl.BlockSpec((1, EMBED), lambda i: (0, 0)),
        ],
        out_specs=[
            pl.BlockSpec((_ROWS_B, EMBED), lambda i: (i, 0)),
            pl.BlockSpec((_ROWS_B, EMBED), lambda i: (i, 0)),
        ],
        out_shape=[
            jax.ShapeDtypeStruct((NPAD, EMBED), jnp.float32),
            jax.ShapeDtypeStruct((NPAD, EMBED), jnp.float32),
        ],
    )(acc_parts, tblprev, dis_b, W, b)


_SB = 8
_TB = _SB * SESS_LEN
_HD = EMBED // HEADS


def _tc_attn_body(x, wqkv, bqkv, wo, bo, out_o):
    xb = x[...]
    qkv = lax.dot_general(xb, wqkv[...], (((1,), (1,)), ((), ())),
                          preferred_element_type=jnp.float32) + bqkv[...]
    r2 = lax.broadcasted_iota(jnp.int32, (_TB, _TB), 0) // SESS_LEN
    c2 = lax.broadcasted_iota(jnp.int32, (_TB, _TB), 1) // SESS_LEN
    same = r2 == c2
    pieces = []
    for h in range(HEADS):
        q = qkv[:, h * _HD:(h + 1) * _HD]
        k = qkv[:, EMBED + h * _HD:EMBED + (h + 1) * _HD]
        v = qkv[:, 2 * EMBED + h * _HD:2 * EMBED + (h + 1) * _HD]
        s = lax.dot_general(q, k, (((1,), (1,)), ((), ())),
                            preferred_element_type=jnp.float32)
        s = s * np.float32(1.0 / np.sqrt(_HD))
        s = jnp.where(same, s, -1e30)
        m = jnp.max(s, axis=1, keepdims=True)
        p = jnp.exp(s - m)
        p = p / jnp.sum(p, axis=1, keepdims=True)
        pieces.append(lax.dot_general(p, v, (((1,), (0,)), ((), ())),
                                      preferred_element_type=jnp.float32))
    ctx = jnp.concatenate(pieces, axis=1)
    ao = lax.dot_general(ctx, wo[...], (((1,), (1,)), ((), ())),
                         preferred_element_type=jnp.float32) + bo[...]
    rm = lax.broadcasted_iota(jnp.int32, (_SB, _TB), 0)
    cm = lax.broadcasted_iota(jnp.int32, (_SB, _TB), 1) // SESS_LEN
    Mm = jnp.where(cm == rm, np.float32(1.0 / SESS_LEN), 0.0)
    out_o[...] = lax.dot_general(Mm, ao, (((1,), (0,)), ((), ())),
                                 preferred_element_type=jnp.float32)


def _tc_attn(geo, wqkv, bqkv, wo, bo):
    return pl.pallas_call(
        _tc_attn_body,
        grid=(N_SESS // _SB,),
        in_specs=[
            pl.BlockSpec((_TB, EMBED), lambda i: (i, 0)),
            pl.BlockSpec((3 * EMBED, EMBED), lambda i: (0, 0)),
            pl.BlockSpec((1, 3 * EMBED), lambda i: (0, 0)),
            pl.BlockSpec((EMBED, EMBED), lambda i: (0, 0)),
            pl.BlockSpec((1, EMBED), lambda i: (0, 0)),
        ],
        out_specs=pl.BlockSpec((_SB, EMBED), lambda i: (i, 0)),
        out_shape=jax.ShapeDtypeStruct((N_SESS, EMBED), jnp.float32),
    )(geo, wqkv, bqkv, wo, bo)


def kernel(poi_embed_table, gcn_W, gcn_b, in_proj_w, in_proj_b, out_proj_w,
           out_proj_b, dist_vec, dist_edges, batch_idx, poi, x_idx):
    i32 = jnp.int32
    f32 = jnp.float32
    pad = EPAD - E_RAW
    trash = jnp.full((pad,), NPAD - 1, i32)
    e0 = jnp.concatenate([dist_edges[0].astype(i32), trash])
    e1 = jnp.concatenate([dist_edges[1].astype(i32), trash])
    dvp = jnp.concatenate([dist_vec, jnp.zeros((pad,), f32)])
    emb_pad = jnp.pad(poi_embed_table, ((0, NPAD - N_POI), (0, 0)))
    zeros128 = jnp.zeros((NPAD, EMBED), f32)
    ones128 = jnp.ones((CHUNK, EMBED), f32)

    w16 = _tc_wexp(dvp.reshape(EPAD, 1))
    deg_parts = _sc_deg(e0, e1, zeros128, ones128)
    dis_b, enc0p = _tc_prep(deg_parts, emb_pad)
    acc1 = _sc_agg(e0, e1, w16, enc0p, zeros128)
    enc1, enc1s = _tc_dense(acc1, enc0p, dis_b, gcn_W[0],
                            gcn_b[0].reshape(1, -1))
    acc2 = _sc_agg(e0, e1, w16, enc1s, zeros128)
    enc2, _ = _tc_dense(acc2, enc1s, dis_b, gcn_W[1], gcn_b[1].reshape(1, -1))

    gidx = jnp.concatenate([poi.astype(i32), x_idx.astype(i32)])
    rows = _sc_gather(gidx, enc2)
    tar_embed = rows[:N_SESS]
    geo = rows[N_SESS:]
    aggr_feat = _tc_attn(geo, in_proj_w, in_proj_b.reshape(1, -1),
                         out_proj_w, out_proj_b.reshape(1, -1))
    return aggr_feat, tar_embed


# speedup vs baseline: 13.8418x; 1.1033x over previous
import functools

import jax
import jax.numpy as jnp
import numpy as np
from jax import lax
from jax.experimental import pallas as pl
from jax.experimental.pallas import tpu as pltpu
from jax.experimental.pallas import tpu_sc as plsc

N_POI = 10000
EMBED = 128
HEADS = 4
E_RAW = 320000
N_SESS = 256
SESS_LEN = 32

NC, NS = 2, 16
NW = NC * NS
NPAD = 10112
ROWS_PER_TILE = NPAD // NS
CHUNK = 128
CPT = -(-E_RAW // (CHUNK * NW))
EPAD = CPT * NW * CHUNK
GIDX = N_SESS + N_SESS * SESS_LEN
GPT = GIDX // NW


def _mesh():
    return plsc.VectorSubcoreMesh(core_axis_name="c", subcore_axis_name="s",
                                  num_cores=NC, num_subcores=NS)


def _stripe(sid):
    return pl.ds(sid * ROWS_PER_TILE, ROWS_PER_TILE)


def _sc_deg_body(e0_hbm, e1_hbm, zeros128_hbm, ones128_hbm, deg_out, acc_sh,
                 idx_v, ones_v):
    cid = lax.axis_index("c")
    sid = lax.axis_index("s")
    wid = sid * NC + cid

    pltpu.sync_copy(zeros128_hbm.at[_stripe(sid)], acc_sh.at[_stripe(sid)])
    pltpu.sync_copy(ones128_hbm, ones_v)
    plsc.subcore_barrier()

    def chunk_body(k, carry):
        base = (wid * CPT + k) * CHUNK
        pltpu.sync_copy(e0_hbm.at[pl.ds(base, CHUNK)], idx_v.at[0])
        pltpu.sync_copy(e1_hbm.at[pl.ds(base, CHUNK)], idx_v.at[1])
        pltpu.sync_copy(ones_v, acc_sh.at[idx_v.at[0]], add=True)
        pltpu.sync_copy(ones_v, acc_sh.at[idx_v.at[1]], add=True)
        return carry

    lax.fori_loop(0, CPT, chunk_body, 0)
    plsc.subcore_barrier()
    pltpu.sync_copy(acc_sh.at[_stripe(sid)], deg_out.at[cid, _stripe(sid)])


@functools.lru_cache(maxsize=None)
def _sc_deg_call():
    return pl.kernel(
        _sc_deg_body,
        out_type=jax.ShapeDtypeStruct((NC, NPAD, EMBED), jnp.float32),
        mesh=_mesh(),
        scratch_types=[
            pltpu.VMEM_SHARED((NPAD, EMBED), jnp.float32),
            pltpu.VMEM((2, CHUNK), jnp.int32),
            pltpu.VMEM((CHUNK, EMBED), jnp.float32),
        ],
    )


def _sc_deg(e0, e1, zeros128, ones128):
    return _sc_deg_call()(e0, e1, zeros128, ones128)


CPT0 = 101
CPT1 = 2 * CPT - CPT0


def _sc_agg_body(e0_hbm, e1_hbm, wflat_hbm, tbl_hbm, zeros128_hbm, acc_out,
                 acc_sh, idx_v, w_v, rows_v, gsem, ssem, isem):
    cid = lax.axis_index("c")
    sid = lax.axis_index("s")

    cpt_c = jnp.where(cid == 0, CPT0, CPT1)
    chunk0 = cid * NS * CPT0 + sid * cpt_c

    pltpu.sync_copy(zeros128_hbm.at[_stripe(sid)], acc_sh.at[_stripe(sid)])
    plsc.subcore_barrier()

    def cp_idx_async(k, b):
        base = (chunk0 + k) * CHUNK
        pltpu.async_copy(e0_hbm.at[pl.ds(base, CHUNK)], idx_v.at[b, 0], isem)
        pltpu.async_copy(e1_hbm.at[pl.ds(base, CHUNK)], idx_v.at[b, 1], isem)
        pltpu.async_copy(wflat_hbm.at[pl.ds(base * 16, CHUNK * 16)],
                         w_v.at[b], isem)

    def drain_idx():
        pltpu.make_async_copy(e0_hbm.at[pl.ds(0, CHUNK)], idx_v.at[0, 0],
                              isem).wait()
        pltpu.make_async_copy(e1_hbm.at[pl.ds(0, CHUNK)], idx_v.at[0, 1],
                              isem).wait()
        pltpu.make_async_copy(wflat_hbm.at[pl.ds(0, CHUNK * 16)], w_v.at[0],
                              isem).wait()

    def issue_gather(s):
        k = lax.div(s, 2)
        d = lax.rem(s, 2)
        pltpu.async_copy(tbl_hbm.at[idx_v.at[lax.rem(k, 2), 1 - d]],
                         rows_v.at[lax.rem(s, 2)], gsem)

    def drain_scatter():
        pltpu.make_async_copy(rows_v.at[0], acc_sh.at[idx_v.at[0, 0]],
                              ssem).wait()

    cp_idx_async(0, 0)
    drain_idx()
    issue_gather(0)

    S = 2 * cpt_c

    def loop_body(s, carry):
        k = lax.div(s, 2)
        d = lax.rem(s, 2)
        b = lax.rem(s, 2)
        bk = lax.rem(k, 2)

        @pl.when(s > 0)
        def _():
            drain_scatter()

        @pl.when((d == 1) & (k + 1 < cpt_c))
        def _():
            drain_idx()

        @pl.when(s + 1 < S)
        def _():
            issue_gather(s + 1)

        pltpu.make_async_copy(tbl_hbm.at[idx_v.at[bk, 1 - d]],
                              rows_v.at[b], gsem).wait()

        def e_body(e, ec):
            sc = w_v[bk, pl.ds(e * 16, 16)]
            for j in range(EMBED // 16):
                sl = pl.ds(j * 16, 16)
                rows_v[b, e, sl] = rows_v[b, e, sl] * sc
            return ec

        lax.fori_loop(0, CHUNK, e_body, 0)
        pltpu.async_copy(rows_v.at[b], acc_sh.at[idx_v.at[bk, d]], ssem,
                         add=True)

        @pl.when((d == 0) & (k + 1 < cpt_c))
        def _():
            cp_idx_async(k + 1, lax.rem(k + 1, 2))

        return carry

    lax.fori_loop(0, S, loop_body, 0)
    drain_scatter()
    plsc.subcore_barrier()
    pltpu.sync_copy(acc_sh.at[_stripe(sid)], acc_out.at[cid, _stripe(sid)])


@functools.lru_cache(maxsize=None)
def _sc_agg_call():
    return pl.kernel(
        _sc_agg_body,
        out_type=jax.ShapeDtypeStruct((NC, NPAD, EMBED), jnp.float32),
        mesh=_mesh(),
        scratch_types=[
            pltpu.VMEM_SHARED((NPAD, EMBED), jnp.float32),
            pltpu.VMEM((2, 2, CHUNK), jnp.int32),
            pltpu.VMEM((2, CHUNK * 16), jnp.float32),
            pltpu.VMEM((2, CHUNK, EMBED), jnp.float32),
            pltpu.SemaphoreType.DMA,
            pltpu.SemaphoreType.DMA,
            pltpu.SemaphoreType.DMA,
        ],
    )


def _sc_agg(e0, e1, w16, tbl, zeros128):
    return _sc_agg_call()(e0, e1, w16.reshape(EPAD * 16), tbl, zeros128)


def _sc_gather_body(idx_hbm, tbl_hbm, rows_out, idx_v, rows_v, sem):
    cid = lax.axis_index("c")
    sid = lax.axis_index("s")
    wid = sid * NC + cid
    for off, cnt in ((0, 128), (128, 128), (256, GPT - 256)):
        base = wid * GPT + off
        pltpu.sync_copy(idx_hbm.at[pl.ds(base, cnt)], idx_v.at[0, pl.ds(0, cnt)])
        pltpu.async_copy(tbl_hbm.at[idx_v.at[0, pl.ds(0, cnt)]],
                         rows_v.at[pl.ds(0, cnt)], sem).wait()
        pltpu.sync_copy(rows_v.at[pl.ds(0, cnt)], rows_out.at[pl.ds(base, cnt)])


@functools.lru_cache(maxsize=None)
def _sc_gather_call():
    return pl.kernel(
        _sc_gather_body,
        out_type=jax.ShapeDtypeStruct((GIDX, EMBED), jnp.float32),
        mesh=_mesh(),
        scratch_types=[
            pltpu.VMEM((1, 128), jnp.int32),
            pltpu.VMEM((128, EMBED), jnp.float32),
            pltpu.SemaphoreType.DMA,
        ],
    )


def _sc_gather(idx, tbl):
    return _sc_gather_call()(idx, tbl)


_WBLK = 4096


def _tc_wexp_body(dv, w_o):
    d = dv[...]
    w_o[...] = jnp.broadcast_to(jnp.exp(-d * d), (_WBLK, 16))


def _tc_wexp(dvp_col):
    return pl.pallas_call(
        _tc_wexp_body,
        grid=(EPAD // _WBLK,),
        in_specs=[pl.BlockSpec((_WBLK, 1), lambda i: (i, 0))],
        out_specs=pl.BlockSpec((_WBLK, 16), lambda i: (i, 0)),
        out_shape=jax.ShapeDtypeStruct((EPAD, 16), jnp.float32),
    )(dvp_col)


_ROWS_B = 632
_NBLK = NPAD // _ROWS_B


def _tc_prep_body(degp, emb, dis_o, encp_o):
    deg = degp[0, :, 0:1] + degp[1, :, 0:1] + 1.0
    dis = lax.rsqrt(deg)
    disb = jnp.broadcast_to(dis, (_ROWS_B, EMBED))
    dis_o[...] = disb
    encp_o[...] = disb * emb[...]


def _tc_prep(deg_parts, emb_pad):
    return pl.pallas_call(
        _tc_prep_body,
        grid=(_NBLK,),
        in_specs=[
            pl.BlockSpec((NC, _ROWS_B, EMBED), lambda i: (0, i, 0)),
            pl.BlockSpec((_ROWS_B, EMBED), lambda i: (i, 0)),
        ],
        out_specs=[
            pl.BlockSpec((_ROWS_B, EMBED), lambda i: (i, 0)),
            pl.BlockSpec((_ROWS_B, EMBED), lambda i: (i, 0)),
        ],
        out_shape=[
            jax.ShapeDtypeStruct((NPAD, EMBED), jnp.float32),
            jax.ShapeDtypeStruct((NPAD, EMBED), jnp.float32),
        ],
    )(deg_parts, emb_pad)


def _tc_dense_body(accp, tblprev, disb, W, b, enc_o, encs_o):
    side = (accp[0] + accp[1] + tblprev[...]) * disb[...]
    out = lax.dot_general(side, W[...], (((1,), (1,)), ((), ())),
                          preferred_element_type=jnp.float32) + b[...]
    out = jnp.where(out >= 0, out, 0.01 * out)
    nrm = jnp.sqrt(jnp.sum(out * out, axis=1, keepdims=True))
    enc = out / jnp.maximum(nrm, 1e-12)
    enc_o[...] = enc
    encs_o[...] = enc * disb[...]


def _tc_dense(acc_parts, tblprev, dis_b, W, b):
    return pl.pallas_call(
        _tc_dense_body,
        grid=(_NBLK,),
        in_specs=[
            pl.BlockSpec((NC, _ROWS_B, EMBED), lambda i: (0, i, 0)),
            pl.BlockSpec((_ROWS_B, EMBED), lambda i: (i, 0)),
            pl.BlockSpec((_ROWS_B, EMBED), lambda i: (i, 0)),
            pl.BlockSpec((EMBED, EMBED), lambda i: (0, 0)),
            pl.BlockSpec((1, EMBED), lambda i: (0, 0)),
        ],
        out_specs=[
            pl.BlockSpec((_ROWS_B, EMBED), lambda i: (i, 0)),
            pl.BlockSpec((_ROWS_B, EMBED), lambda i: (i, 0)),
        ],
        out_shape=[
            jax.ShapeDtypeStruct((NPAD, EMBED), jnp.float32),
            jax.ShapeDtypeStruct((NPAD, EMBED), jnp.float32),
        ],
    )(acc_parts, tblprev, dis_b, W, b)


_SB = 8
_TB = _SB * SESS_LEN
_HD = EMBED // HEADS


def _tc_attn_body(x, wqkv, bqkv, wo, bo, out_o):
    xb = x[...]
    qkv = lax.dot_general(xb, wqkv[...], (((1,), (1,)), ((), ())),
                          preferred_element_type=jnp.float32) + bqkv[...]
    r2 = lax.broadcasted_iota(jnp.int32, (_TB, _TB), 0) // SESS_LEN
    c2 = lax.broadcasted_iota(jnp.int32, (_TB, _TB), 1) // SESS_LEN
    same = r2 == c2
    pieces = []
    for h in range(HEADS):
        q = qkv[:, h * _HD:(h + 1) * _HD]
        k = qkv[:, EMBED + h * _HD:EMBED + (h + 1) * _HD]
        v = qkv[:, 2 * EMBED + h * _HD:2 * EMBED + (h + 1) * _HD]
        s = lax.dot_general(q, k, (((1,), (1,)), ((), ())),
                            preferred_element_type=jnp.float32)
        s = s * np.float32(1.0 / np.sqrt(_HD))
        s = jnp.where(same, s, -1e30)
        m = jnp.max(s, axis=1, keepdims=True)
        p = jnp.exp(s - m)
        p = p / jnp.sum(p, axis=1, keepdims=True)
        pieces.append(lax.dot_general(p, v, (((1,), (0,)), ((), ())),
                                      preferred_element_type=jnp.float32))
    ctx = jnp.concatenate(pieces, axis=1)
    ao = lax.dot_general(ctx, wo[...], (((1,), (1,)), ((), ())),
                         preferred_element_type=jnp.float32) + bo[...]
    rm = lax.broadcasted_iota(jnp.int32, (_SB, _TB), 0)
    cm = lax.broadcasted_iota(jnp.int32, (_SB, _TB), 1) // SESS_LEN
    Mm = jnp.where(cm == rm, np.float32(1.0 / SESS_LEN), 0.0)
    out_o[...] = lax.dot_general(Mm, ao, (((1,), (0,)), ((), ())),
                                 preferred_element_type=jnp.float32)


def _tc_attn(geo, wqkv, bqkv, wo, bo):
    return pl.pallas_call(
        _tc_attn_body,
        grid=(N_SESS // _SB,),
        in_specs=[
            pl.BlockSpec((_TB, EMBED), lambda i: (i, 0)),
            pl.BlockSpec((3 * EMBED, EMBED), lambda i: (0, 0)),
            pl.BlockSpec((1, 3 * EMBED), lambda i: (0, 0)),
            pl.BlockSpec((EMBED, EMBED), lambda i: (0, 0)),
            pl.BlockSpec((1, EMBED), lambda i: (0, 0)),
        ],
        out_specs=pl.BlockSpec((_SB, EMBED), lambda i: (i, 0)),
        out_shape=jax.ShapeDtypeStruct((N_SESS, EMBED), jnp.float32),
    )(geo, wqkv, bqkv, wo, bo)


def kernel(poi_embed_table, gcn_W, gcn_b, in_proj_w, in_proj_b, out_proj_w,
           out_proj_b, dist_vec, dist_edges, batch_idx, poi, x_idx):
    i32 = jnp.int32
    f32 = jnp.float32
    pad = EPAD - E_RAW
    trash = jnp.full((pad,), NPAD - 1, i32)
    e0 = jnp.concatenate([dist_edges[0].astype(i32), trash])
    e1 = jnp.concatenate([dist_edges[1].astype(i32), trash])
    dvp = jnp.concatenate([dist_vec, jnp.zeros((pad,), f32)])
    emb_pad = jnp.pad(poi_embed_table, ((0, NPAD - N_POI), (0, 0)))
    zeros128 = jnp.zeros((NPAD, EMBED), f32)
    ones128 = jnp.ones((CHUNK, EMBED), f32)

    w16 = _tc_wexp(dvp.reshape(EPAD, 1))
    deg_parts = _sc_deg(e0, e1, zeros128, ones128)
    dis_b, enc0p = _tc_prep(deg_parts, emb_pad)
    acc1 = _sc_agg(e0, e1, w16, enc0p, zeros128)
    enc1, enc1s = _tc_dense(acc1, enc0p, dis_b, gcn_W[0],
                            gcn_b[0].reshape(1, -1))
    acc2 = _sc_agg(e0, e1, w16, enc1s, zeros128)
    enc2, _ = _tc_dense(acc2, enc1s, dis_b, gcn_W[1], gcn_b[1].reshape(1, -1))

    gidx = jnp.concatenate([poi.astype(i32), x_idx.astype(i32)])
    rows = _sc_gather(gidx, enc2)
    tar_embed = rows[:N_SESS]
    geo = rows[N_SESS:]
    aggr_feat = _tc_attn(geo, in_proj_w, in_proj_b.reshape(1, -1),
                         out_proj_w, out_proj_b.reshape(1, -1))
    return aggr_feat, tar_embed
